# Initial kernel scaffold; baseline (speedup 1.0000x reference)
import jax
import jax.numpy as jnp
from jax.experimental import pallas as pl


def kernel(x_seq, edge_index_seq, edge_weight_seq, W_sd0, b_sd0, W_ds0, b_ds0, W_sd1, b_sd1, W_ds1, b_ds1, W_ih, W_hh, b_ih, b_hh, W_p, b_p):
    raise NotImplementedError("write your pallas kernel here")


# speedup vs baseline: 6.9382x; 6.9382x over previous
import functools
import jax
import jax.numpy as jnp
from jax import lax
from jax.experimental import pallas as pl
from jax.experimental.pallas import tpu as pltpu
from jax.experimental.pallas import tpu_sc as plsc

B, S, N, D, E, H = 2, 4, 10000, 128, 160000, 128
NSNAP = B * S
NC, NS = 2, 16
NW = NC * NS
CHUNK = 128
NCHUNKS = E // CHUNK
ROWCH = 80
NROWCH = N // ROWCH
F32 = jnp.float32

@functools.lru_cache(maxsize=None)
def _mesh():
    return plsc.VectorSubcoreMesh(core_axis_name="c", subcore_axis_name="s",
                                  num_cores=NC, num_subcores=NS)


def _deg_body(ei_hbm, w_hbm, degp_hbm, srcb, dstb, wb, zb, od_sp, id_sp):
    c = lax.axis_index("c")
    s = lax.axis_index("s")
    wid = s * NC + c
    zero16 = jnp.zeros((16,), F32)

    @pl.loop(0, N, step=16)
    def _zb(j):
        j = pl.multiple_of(j, 16)
        zb[pl.ds(j, 16)] = zero16

    @pl.loop(0, NSNAP)
    def _snap(i):
        @pl.when(s == 0)
        def _zero():
            pltpu.sync_copy(zb, od_sp)
            pltpu.sync_copy(zb, id_sp)

        plsc.subcore_barrier()

        @pl.loop(wid, NCHUNKS, step=NW)
        def _edges(ch):
            off = pl.multiple_of(ch * CHUNK, CHUNK)
            pltpu.sync_copy(ei_hbm.at[i, 0, pl.ds(off, CHUNK)], srcb)
            pltpu.sync_copy(ei_hbm.at[i, 1, pl.ds(off, CHUNK)], dstb)
            pltpu.sync_copy(w_hbm.at[i, pl.ds(off, CHUNK)], wb)
            pltpu.sync_copy(wb, od_sp.at[srcb], add=True)
            pltpu.sync_copy(wb, id_sp.at[dstb], add=True)

        plsc.subcore_barrier()

        @pl.when(s == 0)
        def _wb():
            pltpu.sync_copy(od_sp, degp_hbm.at[i, 0, c])
            pltpu.sync_copy(id_sp, degp_hbm.at[i, 1, c])

        plsc.subcore_barrier()


def _sc_degrees(ei, w):
    return pl.kernel(
        _deg_body,
        out_type=jax.ShapeDtypeStruct((NSNAP, 2, NC, N), F32),
        mesh=_mesh(),
        scratch_types=[
            pltpu.VMEM((CHUNK,), jnp.int32),
            pltpu.VMEM((CHUNK,), jnp.int32),
            pltpu.VMEM((CHUNK,), F32),
            pltpu.VMEM((N,), F32),
            pltpu.VMEM_SHARED((N,), F32),
            pltpu.VMEM_SHARED((N,), F32),
        ],
        compiler_params=pltpu.CompilerParams(needs_layout_passes=False),
        name="sc_degrees",
    )(ei, w)


def _scale_rows(rows, normb, m):
    nv16 = normb[pl.ds(m, 16)]
    for j in range(16):
        nv = jnp.broadcast_to(nv16[j], (16,))
        for v in range(0, H, 16):
            t = rows[m + j, pl.ds(v, 16)]
            rows[m + j, pl.ds(v, 16)] = t * nv


def _make_spmm_body(with_norm):
    def body_norm(ei_hbm, wn_hbm, invdeg_hbm, ysd_hbm, yds_hbm,
                  aggp_hbm, norm_out_hbm,
                  oib, iib, srcb, dstb, normb, rows, acc):
        _spmm_common(ei_hbm, wn_hbm, invdeg_hbm, ysd_hbm, yds_hbm, aggp_hbm,
                     norm_out_hbm, oib, iib, srcb, dstb, normb, rows, rows,
                     acc, True)

    def body_plain(ei_hbm, wn_hbm, invdeg_hbm, ysd_hbm, yds_hbm,
                   aggp_hbm, norm_out_hbm,
                   srcb, dstb, normb, rows_sd, rows_ds, acc):
        _spmm_common(ei_hbm, wn_hbm, invdeg_hbm, ysd_hbm, yds_hbm, aggp_hbm,
                     norm_out_hbm, None, None, srcb, dstb, normb, rows_sd,
                     rows_ds, acc, False)

    return body_norm if with_norm else body_plain


def _spmm_common(ei_hbm, wn_hbm, invdeg_hbm, ysd_hbm, yds_hbm, aggp_hbm,
                 norm_out_hbm, oib, iib, srcb, dstb, normb, rows_sd, rows_ds,
                 acc, with_norm):
    c = lax.axis_index("c")
    s = lax.axis_index("s")
    wid = s * NC + c
    zero16 = jnp.zeros((16,), F32)
    one_buf = rows_sd is rows_ds

    @pl.loop(0, NSNAP)
    def _snap(i):
        @pl.loop(0, ROWCH)
        def _zb(r):
            for v in range(0, H, 16):
                rows_sd[r, pl.ds(v, 16)] = zero16

        @pl.loop(s, NROWCH, step=NS)
        def _zero(rc):
            r0 = rc * ROWCH
            pltpu.sync_copy(rows_sd.at[pl.ds(0, ROWCH)],
                            acc.at[pl.ds(r0, ROWCH)])

        if with_norm:
            pltpu.sync_copy(invdeg_hbm.at[i, 0], oib)
            pltpu.sync_copy(invdeg_hbm.at[i, 1], iib)
        plsc.subcore_barrier()

        @pl.loop(wid, NCHUNKS, step=NW)
        def _edges(ch):
            off = pl.multiple_of(ch * CHUNK, CHUNK)
            pltpu.sync_copy(ei_hbm.at[i, 0, pl.ds(off, CHUNK)], srcb)
            pltpu.sync_copy(ei_hbm.at[i, 1, pl.ds(off, CHUNK)], dstb)
            pltpu.sync_copy(wn_hbm.at[i, pl.ds(off, CHUNK)], normb)
            if with_norm:
                @pl.loop(0, CHUNK, step=16)
                def _norm(m):
                    m = pl.multiple_of(m, 16)
                    sg = srcb[pl.ds(m, 16)]
                    dg = dstb[pl.ds(m, 16)]
                    nv = (plsc.load_gather(oib, [sg]) *
                          plsc.load_gather(iib, [dg]) * normb[pl.ds(m, 16)])
                    normb[pl.ds(m, 16)] = nv

                pltpu.sync_copy(normb, norm_out_hbm.at[i, pl.ds(off, CHUNK)])

            pltpu.sync_copy(ysd_hbm.at[i].at[srcb], rows_sd)

            @pl.loop(0, CHUNK, step=16)
            def _scale_sd(m):
                _scale_rows(rows_sd, normb, pl.multiple_of(m, 16))

            pltpu.sync_copy(rows_sd, acc.at[dstb], add=True)

            pltpu.sync_copy(yds_hbm.at[i].at[dstb], rows_ds)

            @pl.loop(0, CHUNK, step=16)
            def _scale_ds(m):
                _scale_rows(rows_ds, normb, pl.multiple_of(m, 16))

            pltpu.sync_copy(rows_ds, acc.at[srcb], add=True)

        plsc.subcore_barrier()

        @pl.loop(s, NROWCH, step=NS)
        def _wb(rc):
            r0 = rc * ROWCH
            pltpu.sync_copy(acc.at[pl.ds(r0, ROWCH)],
                            rows_sd.at[pl.ds(0, ROWCH)])
            pltpu.sync_copy(rows_sd.at[pl.ds(0, ROWCH)],
                            aggp_hbm.at[c, i, pl.ds(r0, ROWCH)])

        plsc.subcore_barrier()


def _sc_spmm(ei, wn, invdeg, ysd, yds, with_norm):
    out_type = (jax.ShapeDtypeStruct((NC, NSNAP, N, H), F32),
                jax.ShapeDtypeStruct((NSNAP, E), F32))
    if with_norm:
        scratch = [
            pltpu.VMEM((N,), F32),
            pltpu.VMEM((N,), F32),
            pltpu.VMEM((CHUNK,), jnp.int32),
            pltpu.VMEM((CHUNK,), jnp.int32),
            pltpu.VMEM((CHUNK,), F32),
            pltpu.VMEM((CHUNK, H), F32),
            pltpu.VMEM_SHARED((N, H), F32),
        ]
    else:
        scratch = [
            pltpu.VMEM((CHUNK,), jnp.int32),
            pltpu.VMEM((CHUNK,), jnp.int32),
            pltpu.VMEM((CHUNK,), F32),
            pltpu.VMEM((CHUNK, H), F32),
            pltpu.VMEM((CHUNK, H), F32),
            pltpu.VMEM_SHARED((N, H), F32),
        ]
    return pl.kernel(
        _make_spmm_body(with_norm),
        out_type=out_type,
        mesh=_mesh(),
        scratch_types=scratch,
        compiler_params=pltpu.CompilerParams(needs_layout_passes=False),
        name="sc_spmm_norm" if with_norm else "sc_spmm",
    )(ei, wn, invdeg, ysd, yds)


def _invdeg_kernel(degp_ref, out_ref):
    d = jnp.sum(degp_ref[...], axis=2)
    out_ref[...] = jnp.where(d > 0, lax.rsqrt(d), 0.0)


def _tc_invdeg(degp):
    return pl.pallas_call(
        _invdeg_kernel,
        grid=(NSNAP,),
        in_specs=[pl.BlockSpec((1, 2, NC, N), lambda i: (i, 0, 0, 0))],
        out_specs=pl.BlockSpec((1, 2, N), lambda i: (i, 0, 0)),
        out_shape=jax.ShapeDtypeStruct((NSNAP, 2, N), F32),
    )(degp)


def _premul0_kernel(x_ref, wsd_ref, wds_ref, ysd_ref, yds_ref):
    h = x_ref[...]
    ysd_ref[...] = 0.5 * jnp.dot(h, wsd_ref[...], preferred_element_type=F32)
    yds_ref[...] = 0.5 * jnp.dot(h, wds_ref[...], preferred_element_type=F32)


def _tc_premul0(x, wsd, wds, rows=2000):
    m = x.shape[0]
    return pl.pallas_call(
        _premul0_kernel,
        grid=(m // rows,),
        in_specs=[
            pl.BlockSpec((rows, D), lambda i: (i, 0)),
            pl.BlockSpec((D, H), lambda i: (0, 0)),
            pl.BlockSpec((D, H), lambda i: (0, 0)),
        ],
        out_specs=[
            pl.BlockSpec((rows, H), lambda i: (i, 0)),
            pl.BlockSpec((rows, H), lambda i: (i, 0)),
        ],
        out_shape=[jax.ShapeDtypeStruct((m, H), F32),
                   jax.ShapeDtypeStruct((m, H), F32)],
    )(x, wsd, wds)


def _premul1_kernel(a0_ref, a1_ref, bc_ref, wsd_ref, wds_ref, ysd_ref, yds_ref):
    h = a0_ref[...] + a1_ref[...] + bc_ref[...]
    ysd_ref[...] = 0.5 * jnp.dot(h, wsd_ref[...], preferred_element_type=F32)
    yds_ref[...] = 0.5 * jnp.dot(h, wds_ref[...], preferred_element_type=F32)


def _tc_premul1(a0, a1, bc, wsd, wds, rows=2000):
    m = a0.shape[0]
    return pl.pallas_call(
        _premul1_kernel,
        grid=(m // rows,),
        in_specs=[
            pl.BlockSpec((rows, H), lambda i: (i, 0)),
            pl.BlockSpec((rows, H), lambda i: (i, 0)),
            pl.BlockSpec((1, H), lambda i: (0, 0)),
            pl.BlockSpec((H, H), lambda i: (0, 0)),
            pl.BlockSpec((H, H), lambda i: (0, 0)),
        ],
        out_specs=[
            pl.BlockSpec((rows, H), lambda i: (i, 0)),
            pl.BlockSpec((rows, H), lambda i: (i, 0)),
        ],
        out_shape=[jax.ShapeDtypeStruct((m, H), F32),
                   jax.ShapeDtypeStruct((m, H), F32)],
    )(a0, a1, bc, wsd, wds)


def _lstm_kernel(a0_ref, a1_ref, bc_ref, wih_ref, whh_ref, bl_ref,
                 wp_ref, bp_ref, out_ref):
    x2 = a0_ref[...] + a1_ref[...]
    h = None
    cst = None
    for q in range(S):
        xq = x2[:, q * H:(q + 1) * H] + bc_ref[...]
        g = jnp.dot(xq, wih_ref[...], preferred_element_type=F32) + bl_ref[...]
        if h is not None:
            g = g + jnp.dot(h, whh_ref[...], preferred_element_type=F32)
        i_g = jax.nn.sigmoid(g[:, 0 * H:1 * H])
        f_g = jax.nn.sigmoid(g[:, 1 * H:2 * H])
        c_g = jnp.tanh(g[:, 2 * H:3 * H])
        o_g = jax.nn.sigmoid(g[:, 3 * H:4 * H])
        cst = i_g * c_g if cst is None else f_g * cst + i_g * c_g
        h = o_g * jnp.tanh(cst)
    out_ref[...] = jnp.dot(h, wp_ref[...], preferred_element_type=F32) + bp_ref[...]


def _tc_lstm(a0, a1, bc, wih_t, whh_t, bl, wp, bp, rows=1000):
    m = a0.shape[0]
    return pl.pallas_call(
        _lstm_kernel,
        grid=(m // rows,),
        in_specs=[
            pl.BlockSpec((rows, S * H), lambda i: (i, 0)),
            pl.BlockSpec((rows, S * H), lambda i: (i, 0)),
            pl.BlockSpec((1, H), lambda i: (0, 0)),
            pl.BlockSpec((H, 4 * H), lambda i: (0, 0)),
            pl.BlockSpec((H, 4 * H), lambda i: (0, 0)),
            pl.BlockSpec((1, 4 * H), lambda i: (0, 0)),
            pl.BlockSpec((H, H), lambda i: (0, 0)),
            pl.BlockSpec((1, H), lambda i: (0, 0)),
        ],
        out_specs=pl.BlockSpec((rows, H), lambda i: (i, 0)),
        out_shape=jax.ShapeDtypeStruct((m, H), F32),
    )(a0, a1, bc, wih_t, whh_t, bl, wp, bp)


def kernel(x_seq, edge_index_seq, edge_weight_seq, W_sd0, b_sd0, W_ds0, b_ds0,
           W_sd1, b_sd1, W_ds1, b_ds1, W_ih, W_hh, b_ih, b_hh, W_p, b_p):
    ei = edge_index_seq.reshape(NSNAP, 2, E)
    w = edge_weight_seq.reshape(NSNAP, E)
    x2 = x_seq.reshape(NSNAP * N, D)

    degp = _sc_degrees(ei, w)
    invdeg = _tc_invdeg(degp)

    y0sd, y0ds = _tc_premul0(x2, W_sd0, W_ds0)
    aggp1, norm = _sc_spmm(ei, w, invdeg, y0sd.reshape(NSNAP, N, H),
                           y0ds.reshape(NSNAP, N, H), with_norm=True)

    bc1 = (0.5 * (b_sd0 + b_ds0)).reshape(1, H)
    y1sd, y1ds = _tc_premul1(aggp1[0].reshape(NSNAP * N, H),
                             aggp1[1].reshape(NSNAP * N, H),
                             bc1, W_sd1, W_ds1)
    aggp2, _ = _sc_spmm(ei, norm, invdeg, y1sd.reshape(NSNAP, N, H),
                        y1ds.reshape(NSNAP, N, H), with_norm=False)

    bc2 = (0.5 * (b_sd1 + b_ds1)).reshape(1, H)
    bl = (b_ih + b_hh).reshape(1, 4 * H)
    out = _tc_lstm(aggp2[0].reshape(B * N, S * H),
                   aggp2[1].reshape(B * N, S * H),
                   bc2, W_ih.T, W_hh.T, bl, W_p, b_p.reshape(1, H))
    return out.reshape(B, N, H)


# speedup vs baseline: 7.0056x; 1.0097x over previous
import functools
import jax
import jax.numpy as jnp
from jax import lax
from jax.experimental import pallas as pl
from jax.experimental.pallas import tpu as pltpu
from jax.experimental.pallas import tpu_sc as plsc

B, S, N, D, E, H = 2, 4, 10000, 128, 160000, 128
NSNAP = B * S
NC, NS = 2, 16
NW = NC * NS
CHUNK = 128
NCHUNKS = E // CHUNK
ROWCH = 80
NROWCH = N // ROWCH
F32 = jnp.float32

@functools.lru_cache(maxsize=None)
def _mesh():
    return plsc.VectorSubcoreMesh(core_axis_name="c", subcore_axis_name="s",
                                  num_cores=NC, num_subcores=NS)


def _deg_body(ei_hbm, w_hbm, degp_hbm, srcb, dstb, wb, zb, od_sp, id_sp):
    c = lax.axis_index("c")
    s = lax.axis_index("s")
    wid = s * NC + c
    zero16 = jnp.zeros((16,), F32)

    @pl.loop(0, N, step=16)
    def _zb(j):
        j = pl.multiple_of(j, 16)
        zb[pl.ds(j, 16)] = zero16

    @pl.loop(0, NSNAP)
    def _snap(i):
        @pl.when(s == 0)
        def _zero():
            pltpu.sync_copy(zb, od_sp)
            pltpu.sync_copy(zb, id_sp)

        plsc.subcore_barrier()

        @pl.loop(wid, NCHUNKS, step=NW)
        def _edges(ch):
            off = pl.multiple_of(ch * CHUNK, CHUNK)
            pltpu.sync_copy(ei_hbm.at[i, 0, pl.ds(off, CHUNK)], srcb)
            pltpu.sync_copy(ei_hbm.at[i, 1, pl.ds(off, CHUNK)], dstb)
            pltpu.sync_copy(w_hbm.at[i, pl.ds(off, CHUNK)], wb)
            pltpu.sync_copy(wb, od_sp.at[srcb], add=True)
            pltpu.sync_copy(wb, id_sp.at[dstb], add=True)

        plsc.subcore_barrier()

        @pl.when(s == 0)
        def _wb():
            pltpu.sync_copy(od_sp, degp_hbm.at[i, 0, c])
            pltpu.sync_copy(id_sp, degp_hbm.at[i, 1, c])

        plsc.subcore_barrier()


def _sc_degrees(ei, w):
    return pl.kernel(
        _deg_body,
        out_type=jax.ShapeDtypeStruct((NSNAP, 2, NC, N), F32),
        mesh=_mesh(),
        scratch_types=[
            pltpu.VMEM((CHUNK,), jnp.int32),
            pltpu.VMEM((CHUNK,), jnp.int32),
            pltpu.VMEM((CHUNK,), F32),
            pltpu.VMEM((N,), F32),
            pltpu.VMEM_SHARED((N,), F32),
            pltpu.VMEM_SHARED((N,), F32),
        ],
        compiler_params=pltpu.CompilerParams(needs_layout_passes=False),
        name="sc_degrees",
    )(ei, w)


def _norms_body(ei_hbm, w_hbm, invdeg_hbm, norm_hbm, oib, iib, srcb, dstb,
                normb):
    c = lax.axis_index("c")
    s = lax.axis_index("s")
    wid = s * NC + c

    @pl.loop(0, NSNAP)
    def _snap(i):
        pltpu.sync_copy(invdeg_hbm.at[i, 0], oib)
        pltpu.sync_copy(invdeg_hbm.at[i, 1], iib)

        @pl.loop(wid, NCHUNKS, step=NW)
        def _edges(ch):
            off = pl.multiple_of(ch * CHUNK, CHUNK)
            pltpu.sync_copy(ei_hbm.at[i, 0, pl.ds(off, CHUNK)], srcb)
            pltpu.sync_copy(ei_hbm.at[i, 1, pl.ds(off, CHUNK)], dstb)
            pltpu.sync_copy(w_hbm.at[i, pl.ds(off, CHUNK)], normb)

            @plsc.parallel_loop(0, CHUNK, step=16)
            def _norm(m):
                m = pl.multiple_of(m, 16)
                sg = srcb[pl.ds(m, 16)]
                dg = dstb[pl.ds(m, 16)]
                nv = (plsc.load_gather(oib, [sg]) *
                      plsc.load_gather(iib, [dg]) * normb[pl.ds(m, 16)])
                normb[pl.ds(m, 16)] = nv

            pltpu.sync_copy(normb, norm_hbm.at[i, pl.ds(off, CHUNK)])


def _sc_norms(ei, w, invdeg):
    return pl.kernel(
        _norms_body,
        out_type=jax.ShapeDtypeStruct((NSNAP, E), F32),
        mesh=_mesh(),
        scratch_types=[
            pltpu.VMEM((N,), F32),
            pltpu.VMEM((N,), F32),
            pltpu.VMEM((CHUNK,), jnp.int32),
            pltpu.VMEM((CHUNK,), jnp.int32),
            pltpu.VMEM((CHUNK,), F32),
        ],
        compiler_params=pltpu.CompilerParams(needs_layout_passes=False),
        name="sc_norms",
    )(ei, w, invdeg)


def _scale_rows(rows, normb, m):
    nv16 = normb[pl.ds(m, 16)]
    for j in range(16):
        nv = jnp.broadcast_to(nv16[j], (16,))
        for v in range(0, H, 16):
            t = rows[m + j, pl.ds(v, 16)]
            rows[m + j, pl.ds(v, 16)] = t * nv


def _spmm_body(ei_hbm, norm_hbm, ysd_hbm, yds_hbm, aggp_hbm,
               srcb, dstb, normb, rows_sd, rows_ds, acc, sem1, sem2):
    c = lax.axis_index("c")
    s = lax.axis_index("s")
    wid = s * NC + c
    zero16 = jnp.zeros((16,), F32)

    @pl.loop(0, NSNAP)
    def _snap(i):
        @pl.loop(0, ROWCH)
        def _zb(r):
            for v in range(0, H, 16):
                rows_sd[r, pl.ds(v, 16)] = zero16

        @pl.loop(s, NROWCH, step=NS)
        def _zero(rc):
            r0 = rc * ROWCH
            pltpu.sync_copy(rows_sd.at[pl.ds(0, ROWCH)],
                            acc.at[pl.ds(r0, ROWCH)])

        plsc.subcore_barrier()

        @pl.loop(wid, NCHUNKS, step=NW)
        def _edges(ch):
            off = pl.multiple_of(ch * CHUNK, CHUNK)
            pltpu.sync_copy(ei_hbm.at[i, 0, pl.ds(off, CHUNK)], srcb)
            pltpu.sync_copy(ei_hbm.at[i, 1, pl.ds(off, CHUNK)], dstb)
            pltpu.sync_copy(norm_hbm.at[i, pl.ds(off, CHUNK)], normb)

            d1 = pltpu.async_copy(ysd_hbm.at[i].at[srcb], rows_sd, sem1)
            d2 = pltpu.async_copy(yds_hbm.at[i].at[dstb], rows_ds, sem2)
            d1.wait()

            @plsc.parallel_loop(0, CHUNK, step=16)
            def _scale_sd(m):
                _scale_rows(rows_sd, normb, pl.multiple_of(m, 16))

            pltpu.sync_copy(rows_sd, acc.at[dstb], add=True)
            d2.wait()

            @plsc.parallel_loop(0, CHUNK, step=16)
            def _scale_ds(m):
                _scale_rows(rows_ds, normb, pl.multiple_of(m, 16))

            pltpu.sync_copy(rows_ds, acc.at[srcb], add=True)

        plsc.subcore_barrier()

        @pl.loop(s, NROWCH, step=NS)
        def _wb(rc):
            r0 = rc * ROWCH
            pltpu.sync_copy(acc.at[pl.ds(r0, ROWCH)],
                            rows_sd.at[pl.ds(0, ROWCH)])
            pltpu.sync_copy(rows_sd.at[pl.ds(0, ROWCH)],
                            aggp_hbm.at[c, i, pl.ds(r0, ROWCH)])

        plsc.subcore_barrier()


def _sc_spmm(ei, norm, ysd, yds):
    return pl.kernel(
        _spmm_body,
        out_type=jax.ShapeDtypeStruct((NC, NSNAP, N, H), F32),
        mesh=_mesh(),
        scratch_types=[
            pltpu.VMEM((CHUNK,), jnp.int32),
            pltpu.VMEM((CHUNK,), jnp.int32),
            pltpu.VMEM((CHUNK,), F32),
            pltpu.VMEM((CHUNK, H), F32),
            pltpu.VMEM((CHUNK, H), F32),
            pltpu.VMEM_SHARED((N, H), F32),
            pltpu.SemaphoreType.DMA,
            pltpu.SemaphoreType.DMA,
        ],
        compiler_params=pltpu.CompilerParams(needs_layout_passes=False),
        name="sc_spmm",
    )(ei, norm, ysd, yds)


def _invdeg_kernel(degp_ref, out_ref):
    d = jnp.sum(degp_ref[...], axis=2)
    out_ref[...] = jnp.where(d > 0, lax.rsqrt(d), 0.0)


def _tc_invdeg(degp):
    return pl.pallas_call(
        _invdeg_kernel,
        grid=(NSNAP,),
        in_specs=[pl.BlockSpec((1, 2, NC, N), lambda i: (i, 0, 0, 0))],
        out_specs=pl.BlockSpec((1, 2, N), lambda i: (i, 0, 0)),
        out_shape=jax.ShapeDtypeStruct((NSNAP, 2, N), F32),
    )(degp)


def _premul0_kernel(x_ref, wsd_ref, wds_ref, ysd_ref, yds_ref):
    h = x_ref[...]
    ysd_ref[...] = 0.5 * jnp.dot(h, wsd_ref[...], preferred_element_type=F32)
    yds_ref[...] = 0.5 * jnp.dot(h, wds_ref[...], preferred_element_type=F32)


def _tc_premul0(x, wsd, wds, rows=2000):
    m = x.shape[0]
    return pl.pallas_call(
        _premul0_kernel,
        grid=(m // rows,),
        in_specs=[
            pl.BlockSpec((rows, D), lambda i: (i, 0)),
            pl.BlockSpec((D, H), lambda i: (0, 0)),
            pl.BlockSpec((D, H), lambda i: (0, 0)),
        ],
        out_specs=[
            pl.BlockSpec((rows, H), lambda i: (i, 0)),
            pl.BlockSpec((rows, H), lambda i: (i, 0)),
        ],
        out_shape=[jax.ShapeDtypeStruct((m, H), F32),
                   jax.ShapeDtypeStruct((m, H), F32)],
    )(x, wsd, wds)


def _premul1_kernel(a0_ref, a1_ref, bc_ref, wsd_ref, wds_ref, ysd_ref, yds_ref):
    h = a0_ref[...] + a1_ref[...] + bc_ref[...]
    ysd_ref[...] = 0.5 * jnp.dot(h, wsd_ref[...], preferred_element_type=F32)
    yds_ref[...] = 0.5 * jnp.dot(h, wds_ref[...], preferred_element_type=F32)


def _tc_premul1(a0, a1, bc, wsd, wds, rows=2000):
    m = a0.shape[0]
    return pl.pallas_call(
        _premul1_kernel,
        grid=(m // rows,),
        in_specs=[
            pl.BlockSpec((rows, H), lambda i: (i, 0)),
            pl.BlockSpec((rows, H), lambda i: (i, 0)),
            pl.BlockSpec((1, H), lambda i: (0, 0)),
            pl.BlockSpec((H, H), lambda i: (0, 0)),
            pl.BlockSpec((H, H), lambda i: (0, 0)),
        ],
        out_specs=[
            pl.BlockSpec((rows, H), lambda i: (i, 0)),
            pl.BlockSpec((rows, H), lambda i: (i, 0)),
        ],
        out_shape=[jax.ShapeDtypeStruct((m, H), F32),
                   jax.ShapeDtypeStruct((m, H), F32)],
    )(a0, a1, bc, wsd, wds)


def _lstm_kernel(a0_ref, a1_ref, bc_ref, wih_ref, whh_ref, bl_ref,
                 wp_ref, bp_ref, out_ref):
    x2 = a0_ref[...] + a1_ref[...]
    h = None
    cst = None
    for q in range(S):
        xq = x2[:, q * H:(q + 1) * H] + bc_ref[...]
        g = jnp.dot(xq, wih_ref[...], preferred_element_type=F32) + bl_ref[...]
        if h is not None:
            g = g + jnp.dot(h, whh_ref[...], preferred_element_type=F32)
        i_g = jax.nn.sigmoid(g[:, 0 * H:1 * H])
        f_g = jax.nn.sigmoid(g[:, 1 * H:2 * H])
        c_g = jnp.tanh(g[:, 2 * H:3 * H])
        o_g = jax.nn.sigmoid(g[:, 3 * H:4 * H])
        cst = i_g * c_g if cst is None else f_g * cst + i_g * c_g
        h = o_g * jnp.tanh(cst)
    out_ref[...] = jnp.dot(h, wp_ref[...], preferred_element_type=F32) + bp_ref[...]


def _tc_lstm(a0, a1, bc, wih_t, whh_t, bl, wp, bp, rows=1000):
    m = a0.shape[0]
    return pl.pallas_call(
        _lstm_kernel,
        grid=(m // rows,),
        in_specs=[
            pl.BlockSpec((rows, S * H), lambda i: (i, 0)),
            pl.BlockSpec((rows, S * H), lambda i: (i, 0)),
            pl.BlockSpec((1, H), lambda i: (0, 0)),
            pl.BlockSpec((H, 4 * H), lambda i: (0, 0)),
            pl.BlockSpec((H, 4 * H), lambda i: (0, 0)),
            pl.BlockSpec((1, 4 * H), lambda i: (0, 0)),
            pl.BlockSpec((H, H), lambda i: (0, 0)),
            pl.BlockSpec((1, H), lambda i: (0, 0)),
        ],
        out_specs=pl.BlockSpec((rows, H), lambda i: (i, 0)),
        out_shape=jax.ShapeDtypeStruct((m, H), F32),
    )(a0, a1, bc, wih_t, whh_t, bl, wp, bp)


def kernel(x_seq, edge_index_seq, edge_weight_seq, W_sd0, b_sd0, W_ds0, b_ds0,
           W_sd1, b_sd1, W_ds1, b_ds1, W_ih, W_hh, b_ih, b_hh, W_p, b_p):
    ei = edge_index_seq.reshape(NSNAP, 2, E)
    w = edge_weight_seq.reshape(NSNAP, E)
    x2 = x_seq.reshape(NSNAP * N, D)

    degp = _sc_degrees(ei, w)
    invdeg = _tc_invdeg(degp)
    norm = _sc_norms(ei, w, invdeg)

    y0sd, y0ds = _tc_premul0(x2, W_sd0, W_ds0)
    aggp1 = _sc_spmm(ei, norm, y0sd.reshape(NSNAP, N, H),
                     y0ds.reshape(NSNAP, N, H))

    bc1 = (0.5 * (b_sd0 + b_ds0)).reshape(1, H)
    y1sd, y1ds = _tc_premul1(aggp1[0].reshape(NSNAP * N, H),
                             aggp1[1].reshape(NSNAP * N, H),
                             bc1, W_sd1, W_ds1)
    aggp2 = _sc_spmm(ei, norm, y1sd.reshape(NSNAP, N, H),
                     y1ds.reshape(NSNAP, N, H))

    bc2 = (0.5 * (b_sd1 + b_ds1)).reshape(1, H)
    bl = (b_ih + b_hh).reshape(1, 4 * H)
    out = _tc_lstm(aggp2[0].reshape(B * N, S * H),
                   aggp2[1].reshape(B * N, S * H),
                   bc2, W_ih.T, W_hh.T, bl, W_p, b_p.reshape(1, H))
    return out.reshape(B, N, H)


# speedup vs baseline: 9.1346x; 1.3039x over previous
import functools
import jax
import jax.numpy as jnp
from jax import lax
from jax.experimental import pallas as pl
from jax.experimental.pallas import tpu as pltpu
from jax.experimental.pallas import tpu_sc as plsc

B, S, N, D, E, H = 2, 4, 10000, 128, 160000, 128
NSNAP = B * S
NC, NS = 2, 16
NW = NC * NS
CHUNK = 128
NCHUNKS = E // CHUNK
GRP = 5
NGRP = NCHUNKS // GRP
ROWCH = 80
NROWCH = N // ROWCH
F32 = jnp.float32

@functools.lru_cache(maxsize=None)
def _mesh():
    return plsc.VectorSubcoreMesh(core_axis_name="c", subcore_axis_name="s",
                                  num_cores=NC, num_subcores=NS)


def _deg_body(ei_hbm, w_hbm, degp_hbm, srcb, dstb, wb, zb, od_sp, id_sp):
    c = lax.axis_index("c")
    s = lax.axis_index("s")
    wid = s * NC + c
    zero16 = jnp.zeros((16,), F32)

    @pl.loop(0, N, step=16)
    def _zb(j):
        j = pl.multiple_of(j, 16)
        zb[pl.ds(j, 16)] = zero16

    @pl.loop(0, NSNAP)
    def _snap(i):
        @pl.when(s == 0)
        def _zero():
            pltpu.sync_copy(zb, od_sp)
            pltpu.sync_copy(zb, id_sp)

        plsc.subcore_barrier()

        @pl.loop(wid, NGRP, step=NW)
        def _edges(g):
            pltpu.sync_copy(ei_hbm.at[i, 0, g], srcb)
            pltpu.sync_copy(ei_hbm.at[i, 1, g], dstb)
            pltpu.sync_copy(w_hbm.at[i, g], wb)
            for j in range(GRP):
                pltpu.sync_copy(wb.at[j], od_sp.at[srcb.at[j]], add=True)
                pltpu.sync_copy(wb.at[j], id_sp.at[dstb.at[j]], add=True)

        plsc.subcore_barrier()

        @pl.when(s == 0)
        def _wb():
            pltpu.sync_copy(od_sp, degp_hbm.at[i, 0, c])
            pltpu.sync_copy(id_sp, degp_hbm.at[i, 1, c])

        plsc.subcore_barrier()


def _sc_degrees(ei, w):
    return pl.kernel(
        _deg_body,
        out_type=jax.ShapeDtypeStruct((NSNAP, 2, NC, N), F32),
        mesh=_mesh(),
        scratch_types=[
            pltpu.VMEM((GRP, CHUNK), jnp.int32),
            pltpu.VMEM((GRP, CHUNK), jnp.int32),
            pltpu.VMEM((GRP, CHUNK), F32),
            pltpu.VMEM((N,), F32),
            pltpu.VMEM_SHARED((N,), F32),
            pltpu.VMEM_SHARED((N,), F32),
        ],
        compiler_params=pltpu.CompilerParams(needs_layout_passes=False),
        name="sc_degrees",
    )(ei, w)


def _norms_body(ei_hbm, w_hbm, invdeg_hbm, norm_hbm, oib, iib, srcb, dstb,
                normb):
    c = lax.axis_index("c")
    s = lax.axis_index("s")
    wid = s * NC + c

    @pl.loop(0, NSNAP)
    def _snap(i):
        pltpu.sync_copy(invdeg_hbm.at[i, 0], oib)
        pltpu.sync_copy(invdeg_hbm.at[i, 1], iib)

        @pl.loop(wid, NGRP, step=NW)
        def _edges(g):
            pltpu.sync_copy(ei_hbm.at[i, 0, g], srcb)
            pltpu.sync_copy(ei_hbm.at[i, 1, g], dstb)
            pltpu.sync_copy(w_hbm.at[i, g], normb)

            for j in range(GRP):
                @plsc.parallel_loop(0, CHUNK, step=16)
                def _norm(m):
                    m = pl.multiple_of(m, 16)
                    sg = srcb[j, pl.ds(m, 16)]
                    dg = dstb[j, pl.ds(m, 16)]
                    nv = (plsc.load_gather(oib, [sg]) *
                          plsc.load_gather(iib, [dg]) * normb[j, pl.ds(m, 16)])
                    normb[j, pl.ds(m, 16)] = nv

            pltpu.sync_copy(normb, norm_hbm.at[i, g])


def _sc_norms(ei, w, invdeg):
    return pl.kernel(
        _norms_body,
        out_type=jax.ShapeDtypeStruct((NSNAP, NGRP, GRP, CHUNK), F32),
        mesh=_mesh(),
        scratch_types=[
            pltpu.VMEM((N,), F32),
            pltpu.VMEM((N,), F32),
            pltpu.VMEM((GRP, CHUNK), jnp.int32),
            pltpu.VMEM((GRP, CHUNK), jnp.int32),
            pltpu.VMEM((GRP, CHUNK), F32),
        ],
        compiler_params=pltpu.CompilerParams(needs_layout_passes=False),
        name="sc_norms",
    )(ei, w, invdeg)


def _scale_rows(rows, normb, m):
    nv16 = normb[pl.ds(m, 16)]
    for j in range(16):
        nv = jnp.broadcast_to(nv16[j], (16,))
        for v in range(0, H, 16):
            t = rows[m + j, pl.ds(v, 16)]
            rows[m + j, pl.ds(v, 16)] = t * nv


def _spmm_body(ei_hbm, norm_hbm, ysd_hbm, yds_hbm, aggp_hbm,
               srcb, dstb, normb, rows_sd, rows_ds, acc, sem1, sem2):
    c = lax.axis_index("c")
    s = lax.axis_index("s")
    wid = s * NC + c
    zero16 = jnp.zeros((16,), F32)

    @pl.loop(0, NSNAP)
    def _snap(i):
        @pl.loop(0, ROWCH)
        def _zb(r):
            for v in range(0, H, 16):
                rows_sd[r, pl.ds(v, 16)] = zero16

        @pl.loop(s, NROWCH, step=NS)
        def _zero(rc):
            r0 = rc * ROWCH
            pltpu.sync_copy(rows_sd.at[pl.ds(0, ROWCH)],
                            acc.at[pl.ds(r0, ROWCH)])

        plsc.subcore_barrier()

        @pl.loop(wid, NGRP, step=NW)
        def _edges(g):
            pltpu.sync_copy(ei_hbm.at[i, 0, g], srcb)
            pltpu.sync_copy(ei_hbm.at[i, 1, g], dstb)
            pltpu.sync_copy(norm_hbm.at[i, g], normb)

            for j in range(GRP):
                d1 = pltpu.async_copy(ysd_hbm.at[i].at[srcb.at[j]], rows_sd,
                                      sem1)
                d2 = pltpu.async_copy(yds_hbm.at[i].at[dstb.at[j]], rows_ds,
                                      sem2)
                d1.wait()

                @plsc.parallel_loop(0, CHUNK, step=16)
                def _scale_sd(m):
                    _scale_rows(rows_sd, normb.at[j], pl.multiple_of(m, 16))

                pltpu.sync_copy(rows_sd, acc.at[dstb.at[j]], add=True)
                d2.wait()

                @plsc.parallel_loop(0, CHUNK, step=16)
                def _scale_ds(m):
                    _scale_rows(rows_ds, normb.at[j], pl.multiple_of(m, 16))

                pltpu.sync_copy(rows_ds, acc.at[srcb.at[j]], add=True)

        plsc.subcore_barrier()

        @pl.loop(s, NROWCH, step=NS)
        def _wb(rc):
            r0 = rc * ROWCH
            pltpu.sync_copy(acc.at[pl.ds(r0, ROWCH)],
                            rows_sd.at[pl.ds(0, ROWCH)])
            pltpu.sync_copy(rows_sd.at[pl.ds(0, ROWCH)],
                            aggp_hbm.at[c, i, pl.ds(r0, ROWCH)])

        plsc.subcore_barrier()


def _sc_spmm(ei, norm, ysd, yds):
    return pl.kernel(
        _spmm_body,
        out_type=jax.ShapeDtypeStruct((NC, NSNAP, N, H), F32),
        mesh=_mesh(),
        scratch_types=[
            pltpu.VMEM((GRP, CHUNK), jnp.int32),
            pltpu.VMEM((GRP, CHUNK), jnp.int32),
            pltpu.VMEM((GRP, CHUNK), F32),
            pltpu.VMEM((CHUNK, H), F32),
            pltpu.VMEM((CHUNK, H), F32),
            pltpu.VMEM_SHARED((N, H), F32),
            pltpu.SemaphoreType.DMA,
            pltpu.SemaphoreType.DMA,
        ],
        compiler_params=pltpu.CompilerParams(needs_layout_passes=False),
        name="sc_spmm",
    )(ei, norm, ysd, yds)


def _invdeg_kernel(degp_ref, out_ref):
    d = jnp.sum(degp_ref[...], axis=2)
    out_ref[...] = jnp.where(d > 0, lax.rsqrt(d), 0.0)


def _tc_invdeg(degp):
    return pl.pallas_call(
        _invdeg_kernel,
        grid=(NSNAP,),
        in_specs=[pl.BlockSpec((1, 2, NC, N), lambda i: (i, 0, 0, 0))],
        out_specs=pl.BlockSpec((1, 2, N), lambda i: (i, 0, 0)),
        out_shape=jax.ShapeDtypeStruct((NSNAP, 2, N), F32),
    )(degp)


def _premul0_kernel(x_ref, wsd_ref, wds_ref, ysd_ref, yds_ref):
    h = x_ref[...]
    ysd_ref[...] = 0.5 * jnp.dot(h, wsd_ref[...], preferred_element_type=F32)
    yds_ref[...] = 0.5 * jnp.dot(h, wds_ref[...], preferred_element_type=F32)


def _tc_premul0(x, wsd, wds, rows=2000):
    m = x.shape[0]
    return pl.pallas_call(
        _premul0_kernel,
        grid=(m // rows,),
        in_specs=[
            pl.BlockSpec((rows, D), lambda i: (i, 0)),
            pl.BlockSpec((D, H), lambda i: (0, 0)),
            pl.BlockSpec((D, H), lambda i: (0, 0)),
        ],
        out_specs=[
            pl.BlockSpec((rows, H), lambda i: (i, 0)),
            pl.BlockSpec((rows, H), lambda i: (i, 0)),
        ],
        out_shape=[jax.ShapeDtypeStruct((m, H), F32),
                   jax.ShapeDtypeStruct((m, H), F32)],
    )(x, wsd, wds)


def _premul1_kernel(a0_ref, a1_ref, bc_ref, wsd_ref, wds_ref, ysd_ref, yds_ref):
    h = a0_ref[...] + a1_ref[...] + bc_ref[...]
    ysd_ref[...] = 0.5 * jnp.dot(h, wsd_ref[...], preferred_element_type=F32)
    yds_ref[...] = 0.5 * jnp.dot(h, wds_ref[...], preferred_element_type=F32)


def _tc_premul1(a0, a1, bc, wsd, wds, rows=2000):
    m = a0.shape[0]
    return pl.pallas_call(
        _premul1_kernel,
        grid=(m // rows,),
        in_specs=[
            pl.BlockSpec((rows, H), lambda i: (i, 0)),
            pl.BlockSpec((rows, H), lambda i: (i, 0)),
            pl.BlockSpec((1, H), lambda i: (0, 0)),
            pl.BlockSpec((H, H), lambda i: (0, 0)),
            pl.BlockSpec((H, H), lambda i: (0, 0)),
        ],
        out_specs=[
            pl.BlockSpec((rows, H), lambda i: (i, 0)),
            pl.BlockSpec((rows, H), lambda i: (i, 0)),
        ],
        out_shape=[jax.ShapeDtypeStruct((m, H), F32),
                   jax.ShapeDtypeStruct((m, H), F32)],
    )(a0, a1, bc, wsd, wds)


def _lstm_kernel(a0_ref, a1_ref, bc_ref, wih_ref, whh_ref, bl_ref,
                 wp_ref, bp_ref, out_ref):
    x2 = a0_ref[...] + a1_ref[...]
    h = None
    cst = None
    for q in range(S):
        xq = x2[:, q * H:(q + 1) * H] + bc_ref[...]
        g = jnp.dot(xq, wih_ref[...], preferred_element_type=F32) + bl_ref[...]
        if h is not None:
            g = g + jnp.dot(h, whh_ref[...], preferred_element_type=F32)
        i_g = jax.nn.sigmoid(g[:, 0 * H:1 * H])
        f_g = jax.nn.sigmoid(g[:, 1 * H:2 * H])
        c_g = jnp.tanh(g[:, 2 * H:3 * H])
        o_g = jax.nn.sigmoid(g[:, 3 * H:4 * H])
        cst = i_g * c_g if cst is None else f_g * cst + i_g * c_g
        h = o_g * jnp.tanh(cst)
    out_ref[...] = jnp.dot(h, wp_ref[...], preferred_element_type=F32) + bp_ref[...]


def _tc_lstm(a0, a1, bc, wih_t, whh_t, bl, wp, bp, rows=1000):
    m = a0.shape[0]
    return pl.pallas_call(
        _lstm_kernel,
        grid=(m // rows,),
        in_specs=[
            pl.BlockSpec((rows, S * H), lambda i: (i, 0)),
            pl.BlockSpec((rows, S * H), lambda i: (i, 0)),
            pl.BlockSpec((1, H), lambda i: (0, 0)),
            pl.BlockSpec((H, 4 * H), lambda i: (0, 0)),
            pl.BlockSpec((H, 4 * H), lambda i: (0, 0)),
            pl.BlockSpec((1, 4 * H), lambda i: (0, 0)),
            pl.BlockSpec((H, H), lambda i: (0, 0)),
            pl.BlockSpec((1, H), lambda i: (0, 0)),
        ],
        out_specs=pl.BlockSpec((rows, H), lambda i: (i, 0)),
        out_shape=jax.ShapeDtypeStruct((m, H), F32),
    )(a0, a1, bc, wih_t, whh_t, bl, wp, bp)


def kernel(x_seq, edge_index_seq, edge_weight_seq, W_sd0, b_sd0, W_ds0, b_ds0,
           W_sd1, b_sd1, W_ds1, b_ds1, W_ih, W_hh, b_ih, b_hh, W_p, b_p):
    ei = edge_index_seq.reshape(NSNAP, 2, NGRP, GRP, CHUNK)
    w = edge_weight_seq.reshape(NSNAP, NGRP, GRP, CHUNK)
    x2 = x_seq.reshape(NSNAP * N, D)

    degp = _sc_degrees(ei, w)
    invdeg = _tc_invdeg(degp)
    norm = _sc_norms(ei, w, invdeg)

    y0sd, y0ds = _tc_premul0(x2, W_sd0, W_ds0)
    aggp1 = _sc_spmm(ei, norm, y0sd.reshape(NSNAP, N, H),
                     y0ds.reshape(NSNAP, N, H))

    bc1 = (0.5 * (b_sd0 + b_ds0)).reshape(1, H)
    y1sd, y1ds = _tc_premul1(aggp1[0].reshape(NSNAP * N, H),
                             aggp1[1].reshape(NSNAP * N, H),
                             bc1, W_sd1, W_ds1)
    aggp2 = _sc_spmm(ei, norm, y1sd.reshape(NSNAP, N, H),
                     y1ds.reshape(NSNAP, N, H))

    bc2 = (0.5 * (b_sd1 + b_ds1)).reshape(1, H)
    bl = (b_ih + b_hh).reshape(1, 4 * H)
    out = _tc_lstm(aggp2[0].reshape(B * N, S * H),
                   aggp2[1].reshape(B * N, S * H),
                   bc2, W_ih.T, W_hh.T, bl, W_p, b_p.reshape(1, H))
    return out.reshape(B, N, H)
